# baseline (speedup 1.0000x reference)
import functools

import jax
import jax.numpy as jnp
from jax import lax
from jax.experimental import pallas as pl
from jax.experimental.pallas import tpu as pltpu
from jax.experimental.pallas import tpu_sc as plsc

N, S, D = 12, 2048, 64
NC, NS, L = 2, 16, 16
NW = NC * NS
R = 16
TASKS1 = N * D
NBLK = N * (S // R)

_mesh = plsc.VectorSubcoreMesh(
    core_axis_name="c", subcore_axis_name="s", num_cores=NC, num_subcores=NS
)


def _sort_body(q_ref, k_ref, qi_ref, ki_ref, v_ref):
    c0 = jnp.concatenate([q_ref[0], k_ref[0]], axis=1)
    p0 = lax.broadcasted_iota(jnp.int32, (S, 2 * D), 0)
    i1 = lax.broadcasted_iota(jnp.int32, (S, 1), 0)

    def make_branch(j):
        def br(args):
            c, p, kk = args
            low = (i1 & j) == 0
            keep_min = jnp.logical_not(jnp.logical_xor((i1 & kk) == 0, low))
            cp = jnp.where(low, pltpu.roll(c, S - j, 0), pltpu.roll(c, j, 0))
            pp = jnp.where(low, pltpu.roll(p, S - j, 0), pltpu.roll(p, j, 0))
            cmp = (c < cp) | ((c == cp) & (p < pp))
            take_own = jnp.logical_not(jnp.logical_xor(cmp, keep_min))
            return jnp.where(take_own, c, cp), jnp.where(take_own, p, pp), kk
        return br

    branches = [make_branch(1 << e) for e in range(11)]

    def stage(_, carry):
        c, p, lk, e = carry
        kk = jnp.int32(1) << lk
        c, p, _ = lax.switch(e, branches, (c, p, kk))
        nlk = jnp.where(e == 0, lk + 1, lk)
        ne = jnp.where(e == 0, lk, e - 1)
        return c, p, nlk, ne

    c, p, _, _ = lax.fori_loop(0, 66, stage, (c0, p0, jnp.int32(1), jnp.int32(0)))
    dqk = c[:, :D] - c[:, D:]
    v = jnp.exp(-(dqk * dqk)) * (1.0 / D)
    qi_ref[0] = p[:, :D].T
    ki_ref[0] = p[:, D:].T
    v_ref[0] = v.T


_sortk = pl.pallas_call(
    _sort_body,
    grid=(N,),
    in_specs=[
        pl.BlockSpec((1, S, D), lambda n: (n, 0, 0)),
        pl.BlockSpec((1, S, D), lambda n: (n, 0, 0)),
    ],
    out_specs=[
        pl.BlockSpec((1, D, S), lambda n: (n, 0, 0)),
        pl.BlockSpec((1, D, S), lambda n: (n, 0, 0)),
        pl.BlockSpec((1, D, S), lambda n: (n, 0, 0)),
    ],
    out_shape=[
        jax.ShapeDtypeStruct((N, D, S), jnp.int32),
        jax.ShapeDtypeStruct((N, D, S), jnp.int32),
        jax.ShapeDtypeStruct((N, D, S), jnp.float32),
    ],
)


@functools.partial(
    pl.kernel,
    out_type=(
        jax.ShapeDtypeStruct((N, D, S), jnp.int32),
        jax.ShapeDtypeStruct((N, D, S), jnp.float32),
    ),
    mesh=_mesh,
    compiler_params=pltpu.CompilerParams(needs_layout_passes=False, use_tc_tiling_on_sc=False),
    scratch_types=[
        pltpu.VMEM((S,), jnp.int32),
        pltpu.VMEM((S,), jnp.int32),
        pltpu.VMEM((S,), jnp.float32),
        pltpu.VMEM((S,), jnp.int32),
        pltpu.VMEM((S,), jnp.float32),
    ],
)
def _stage1(qiT, kiT, viT, colsT, valsT, qir, kir, vir, cr, vr):
    w = lax.axis_index("s") * NC + lax.axis_index("c")
    per = TASKS1 // NW

    def task(t, carry):
        g = w * per + t
        n = g // D
        di = g % D
        pltpu.sync_copy(qiT.at[n, di], qir)
        pltpu.sync_copy(kiT.at[n, di], kir)
        pltpu.sync_copy(viT.at[n, di], vir)

        def grp(j, carry2):
            qi = qir[pl.ds(j * L, L)]
            ki = kir[pl.ds(j * L, L)]
            vv = vir[pl.ds(j * L, L)]
            plsc.store_scatter(cr, [qi], ki)
            plsc.store_scatter(vr, [qi], vv)
            return carry2

        lax.fori_loop(0, S // L, grp, 0)
        pltpu.sync_copy(cr, colsT.at[n, di])
        pltpu.sync_copy(vr, valsT.at[n, di])
        return carry

    lax.fori_loop(0, per, task, 0)


@functools.partial(
    pl.kernel,
    out_type=jax.ShapeDtypeStruct((N, S, S), jnp.float32),
    mesh=_mesh,
    compiler_params=pltpu.CompilerParams(needs_layout_passes=False, use_tc_tiling_on_sc=False),
    scratch_types=[
        pltpu.VMEM((2, R, S), jnp.float32),
        pltpu.VMEM((R, S // 4), jnp.int32),
        pltpu.VMEM((2, D, R), jnp.int32),
        pltpu.VMEM((2, D, R), jnp.float32),
        pltpu.SemaphoreType.DMA,
        pltpu.SemaphoreType.DMA,
    ],
)
def _stage2(colsT, valsT, maskW, out, p_v, m_v, c_v, v_v, sem0, sem1):
    w = lax.axis_index("s") * NC + lax.axis_index("c")
    per = NBLK // NW
    zeros = jnp.zeros((L,), jnp.float32)
    rows_base = lax.iota(jnp.int32, L)
    sems = (sem0, sem1)

    def zrow(r, carry):
        def zcol(cg, carry2):
            p_v[r // R, r % R, pl.ds(cg * L, L)] = zeros
            return carry2

        lax.fori_loop(0, S // L, zcol, 0)
        return carry

    lax.fori_loop(0, 2 * R, zrow, 0)

    def blk2(t2, carry):
        for b in range(2):
            t = t2 * 2 + b
            g = w * per + t
            n = g // (S // R)
            i0 = (g % (S // R)) * R
            pb = p_v.at[b]
            cb = c_v.at[b]
            vb = v_v.at[b]

            @pl.when(t2 >= 1)
            def _wait_and_rezero():
                pltpu.make_async_copy(pb, out.at[n, pl.ds(i0, R)], sems[b]).wait()

                def rezero(it, carry2):
                    cc = cb[it, pl.ds(0, L)]
                    plsc.store_scatter(pb, [rows_base, cc], zeros)
                    return carry2

                lax.fori_loop(0, D, rezero, 0)

            pltpu.sync_copy(maskW.at[n, pl.ds(i0, R)], m_v)
            pltpu.sync_copy(colsT.at[n, :, pl.ds(i0, R)], cb)
            pltpu.sync_copy(valsT.at[n, :, pl.ds(i0, R)], vb)

            def scat(it, carry2):
                cc = cb[it, pl.ds(0, L)]
                vv = vb[it, pl.ds(0, L)]
                word = plsc.load_gather(m_v, [rows_base, cc >> 2])
                keep = ((word >> ((cc & 3) * 8)) & 1) == 0
                plsc.addupdate_scatter(pb, [rows_base, cc], vv, mask=keep)
                return carry2

            lax.fori_loop(0, D, scat, 0)
            pltpu.async_copy(pb, out.at[n, pl.ds(i0, R)], sems[b])
        return carry

    lax.fori_loop(0, per // 2, blk2, 0)
    gl = w * per
    nl = gl // (S // R)
    il = (gl % (S // R)) * R
    pltpu.make_async_copy(p_v.at[0], out.at[nl, pl.ds(il, R)], sem0).wait()
    pltpu.make_async_copy(p_v.at[1], out.at[nl, pl.ds(il, R)], sem1).wait()


def kernel(q, k, attn_mask):
    mask_shape = attn_mask.shape
    qf = q.reshape(N, S, D)
    kf = k.reshape(N, S, D)
    mu8 = attn_mask.reshape(N, S, S // 4, 4).astype(jnp.uint8)
    maskW = lax.bitcast_convert_type(mu8, jnp.int32)
    qiT, kiT, viT = _sortk(qf, kf)
    colsT, valsT = _stage1(qiT, kiT, viT)
    out = _stage2(colsT, valsT, maskW)
    return out.reshape(mask_shape)


# speedup vs baseline: 1.8498x; 1.8498x over previous
import functools

import jax
import jax.numpy as jnp
from jax import lax
from jax.experimental import pallas as pl
from jax.experimental.pallas import tpu as pltpu
from jax.experimental.pallas import tpu_sc as plsc

N, S, D = 12, 2048, 64
NC, NS, L = 2, 16, 16
NW = NC * NS
R = 16
TASKS1 = N * D
NBLK = N * (S // R)

_mesh = plsc.VectorSubcoreMesh(
    core_axis_name="c", subcore_axis_name="s", num_cores=NC, num_subcores=NS
)


@functools.partial(
    pl.kernel,
    out_type=(
        jax.ShapeDtypeStruct((N, D, S), jnp.int32),
        jax.ShapeDtypeStruct((N, D, S), jnp.float32),
    ),
    mesh=_mesh,
    compiler_params=pltpu.CompilerParams(needs_layout_passes=False, use_tc_tiling_on_sc=False),
    scratch_types=[
        pltpu.VMEM((S,), jnp.float32),
        pltpu.VMEM((S,), jnp.float32),
        pltpu.VMEM((S,), jnp.int32),
        pltpu.VMEM((S,), jnp.int32),
        pltpu.VMEM((S,), jnp.int32),
        pltpu.VMEM((S,), jnp.float32),
    ],
)
def _stage1(qT, kT, qiT, kiT, colsT, valsT, qr, kr, qir, kir, cr, vr):
    w = lax.axis_index("s") * NC + lax.axis_index("c")
    per = TASKS1 // NW

    def task(t, carry):
        g = w * per + t
        n = g // D
        di = g % D
        pltpu.sync_copy(qT.at[n, di], qr)
        pltpu.sync_copy(kT.at[n, di], kr)
        pltpu.sync_copy(qiT.at[n, di], qir)
        pltpu.sync_copy(kiT.at[n, di], kir)

        def grp(j, carry2):
            qi = qir[pl.ds(j * L, L)]
            ki = kir[pl.ds(j * L, L)]
            qs = plsc.load_gather(qr, [qi])
            ks = plsc.load_gather(kr, [ki])
            dqk = qs - ks
            v = jnp.exp(-(dqk * dqk)) * (1.0 / D)
            plsc.store_scatter(cr, [qi], ki)
            plsc.store_scatter(vr, [qi], v)
            return carry2

        lax.fori_loop(0, S // L, grp, 0)
        pltpu.sync_copy(cr, colsT.at[n, di])
        pltpu.sync_copy(vr, valsT.at[n, di])
        return carry

    lax.fori_loop(0, per, task, 0)


@functools.partial(
    pl.kernel,
    out_type=jax.ShapeDtypeStruct((N, S, S), jnp.float32),
    mesh=_mesh,
    compiler_params=pltpu.CompilerParams(needs_layout_passes=False, use_tc_tiling_on_sc=False),
    scratch_types=[
        pltpu.VMEM((2, R, S), jnp.float32),
        pltpu.VMEM((R, S // 4), jnp.int32),
        pltpu.VMEM((2, D, R), jnp.int32),
        pltpu.VMEM((2, D, R), jnp.float32),
        pltpu.SemaphoreType.DMA,
        pltpu.SemaphoreType.DMA,
    ],
)
def _stage2(colsT, valsT, maskW, out, p_v, m_v, c_v, v_v, sem0, sem1):
    w = lax.axis_index("s") * NC + lax.axis_index("c")
    per = NBLK // NW
    zeros = jnp.zeros((L,), jnp.float32)
    rows_base = lax.iota(jnp.int32, L)
    sems = (sem0, sem1)

    def zrow(r, carry):
        def zcol(cg, carry2):
            p_v[r // R, r % R, pl.ds(cg * L, L)] = zeros
            return carry2

        lax.fori_loop(0, S // L, zcol, 0)
        return carry

    lax.fori_loop(0, 2 * R, zrow, 0)

    def blk2(t2, carry):
        for b in range(2):
            t = t2 * 2 + b
            g = w * per + t
            n = g // (S // R)
            i0 = (g % (S // R)) * R
            pb = p_v.at[b]
            cb = c_v.at[b]
            vb = v_v.at[b]

            @pl.when(t2 >= 1)
            def _wait_and_rezero():
                pltpu.make_async_copy(pb, out.at[n, pl.ds(i0, R)], sems[b]).wait()

                def rezero(it, carry2):
                    cc = cb[it, pl.ds(0, L)]
                    plsc.store_scatter(pb, [rows_base, cc], zeros)
                    return carry2

                lax.fori_loop(0, D, rezero, 0)

            pltpu.sync_copy(maskW.at[n, pl.ds(i0, R)], m_v)
            pltpu.sync_copy(colsT.at[n, :, pl.ds(i0, R)], cb)
            pltpu.sync_copy(valsT.at[n, :, pl.ds(i0, R)], vb)

            def scat(it, carry2):
                cc = cb[it, pl.ds(0, L)]
                vv = vb[it, pl.ds(0, L)]
                word = plsc.load_gather(m_v, [rows_base, cc >> 2])
                keep = ((word >> ((cc & 3) * 8)) & 1) == 0
                plsc.addupdate_scatter(pb, [rows_base, cc], vv, mask=keep)
                return carry2

            lax.fori_loop(0, D, scat, 0)
            pltpu.async_copy(pb, out.at[n, pl.ds(i0, R)], sems[b])
        return carry

    lax.fori_loop(0, per // 2, blk2, 0)
    gl = w * per
    nl = gl // (S // R)
    il = (gl % (S // R)) * R
    pltpu.make_async_copy(p_v.at[0], out.at[nl, pl.ds(il, R)], sem0).wait()
    pltpu.make_async_copy(p_v.at[1], out.at[nl, pl.ds(il, R)], sem1).wait()


def kernel(q, k, attn_mask):
    mask_shape = attn_mask.shape
    qT = q.reshape(N, S, D).transpose(0, 2, 1)
    kT = k.reshape(N, S, D).transpose(0, 2, 1)
    qiT = jnp.argsort(qT, axis=2).astype(jnp.int32)
    kiT = jnp.argsort(kT, axis=2).astype(jnp.int32)
    mu8 = attn_mask.reshape(N, S, S // 4, 4).astype(jnp.uint8)
    maskW = lax.bitcast_convert_type(mu8, jnp.int32)
    colsT, valsT = _stage1(qT, kT, qiT, kiT)
    out = _stage2(colsT, valsT, maskW)
    return out.reshape(mask_shape)
